# Initial kernel scaffold; baseline (speedup 1.0000x reference)
import functools

import jax
import jax.numpy as jnp
from jax import lax
from jax.experimental import pallas as pl
from jax.experimental.pallas import tpu as pltpu
from jax.experimental.pallas import tpu_sc as plsc

N = 10000
E = 320000
D = 128
R = 8
P = 8192

DP = 144
C = 1000
NCH = N // C
SEG = C * R
ACC_ROWS = 8064
K = 128

_NS = 16
_NC = 2
E_T = E // _NS
CAP = E_T + 224


def _sc_agg_body(xa_hbm, src_hbm, dst_hbm, et_hbm, out_hbm,
                 src_v, seg_v, csrc, cseg, srcrow, segrow, rows, zbuf,
                 acc, sem):
    c = lax.axis_index("c")
    s = lax.axis_index("s")
    base = s * E_T

    pltpu.sync_copy(dst_hbm.at[pl.ds(base, E_T)], seg_v)
    pltpu.sync_copy(et_hbm.at[pl.ds(base, E_T)], csrc.at[pl.ds(0, E_T)])
    pltpu.sync_copy(src_hbm.at[pl.ds(base, E_T)], src_v)

    def _fuse(i, _):
        sl = pl.ds(i * 16, 16)
        seg_v[sl] = seg_v[sl] * R + csrc[sl]
        return 0
    lax.fori_loop(0, E_T // 16, _fuse, 0)

    z16 = jnp.zeros((16,), jnp.float32)

    def _zb(i, _):
        zbuf[i // 9, pl.ds((i % 9) * 16, 16)] = z16
        return 0
    lax.fori_loop(0, K * (DP // 16), _zb, 0)

    dumpv = jnp.arange(16, dtype=jnp.int32) + SEG
    zi16 = jnp.zeros((16,), jnp.int32)

    def _chunk(ch, _):
        chunk = ch * _NC + c
        lo = chunk * SEG

        zb = s * (ACC_ROWS // _NS)
        for j in range(3):
            pltpu.sync_copy(zbuf, acc.at[pl.ds(zb + j * K, K)])
        pltpu.sync_copy(zbuf.at[pl.ds(0, 120)], acc.at[pl.ds(zb + 3 * K, 120)])
        plsc.subcore_barrier()

        def _scan(i, ptr):
            sl = pl.ds(i * 16, 16)
            sv = seg_v[sl]
            m = (sv >= lo) & (sv < lo + SEG)
            plsc.store_compressed(csrc.at[pl.ds(ptr, 16)], src_v[sl], m)
            plsc.store_compressed(cseg.at[pl.ds(ptr, 16)], sv - lo, m)
            return ptr + plsc.all_reduce_population_count(m)[0]
        ptr = lax.fori_loop(0, E_T // 16, _scan, 0)

        for j in range(K // 16):
            csrc[pl.ds(ptr + j * 16, 16)] = zi16
            cseg[pl.ds(ptr + j * 16, 16)] = dumpv

        def _batch(b, _):
            pltpu.sync_copy(csrc.at[pl.ds(b * K, K)], srcrow)
            pltpu.sync_copy(cseg.at[pl.ds(b * K, K)], segrow)
            pltpu.async_copy(xa_hbm.at[srcrow], rows, sem).wait()
            pltpu.sync_copy(rows, acc.at[segrow], add=True)
            return 0
        lax.fori_loop(0, (ptr + K - 1) // K, _batch, 0)
        plsc.subcore_barrier()

        wb = s * (SEG // _NS)
        for j in range(3):
            pltpu.sync_copy(acc.at[pl.ds(wb + j * K, K)],
                            out_hbm.at[pl.ds(lo + wb + j * K, K)])
        pltpu.sync_copy(acc.at[pl.ds(wb + 3 * K, 116)],
                        out_hbm.at[pl.ds(lo + wb + 3 * K, 116)])
        plsc.subcore_barrier()
        return 0

    lax.fori_loop(0, NCH // _NC, _chunk, 0)


_sc_aggregate = functools.partial(
    pl.kernel,
    out_type=jax.ShapeDtypeStruct((N * R, DP), jnp.float32),
    mesh=plsc.VectorSubcoreMesh(core_axis_name="c", subcore_axis_name="s"),
    scratch_types=[
        pltpu.VMEM((E_T,), jnp.int32),
        pltpu.VMEM((E_T,), jnp.int32),
        pltpu.VMEM((CAP,), jnp.int32),
        pltpu.VMEM((CAP,), jnp.int32),
        pltpu.VMEM((K,), jnp.int32),
        pltpu.VMEM((K,), jnp.int32),
        pltpu.VMEM((K, DP), jnp.float32),
        pltpu.VMEM((K, DP), jnp.float32),
        pltpu.VMEM_SHARED((ACC_ROWS, DP), jnp.float32),
        pltpu.SemaphoreType.DMA,
    ],
)(_sc_agg_body)


def _sc_gather_body(tab_hbm, idx_hbm, out_hbm, idx_v, rows_v, sem):
    wid = lax.axis_index("s") * _NC + lax.axis_index("c")
    b_per_w = P // (_NC * _NS)
    base = wid * b_per_w
    pltpu.sync_copy(idx_hbm.at[pl.ds(base, b_per_w)], idx_v)
    pltpu.async_copy(tab_hbm.at[idx_v], rows_v, sem).wait()
    pltpu.sync_copy(rows_v, out_hbm.at[pl.ds(base, b_per_w)])


_sc_gather = functools.partial(
    pl.kernel,
    out_type=jax.ShapeDtypeStruct((P, D), jnp.float32),
    mesh=plsc.VectorSubcoreMesh(core_axis_name="c", subcore_axis_name="s"),
    scratch_types=[
        pltpu.VMEM((P // (_NC * _NS),), jnp.int32),
        pltpu.VMEM((P // (_NC * _NS), D), jnp.float32),
        pltpu.SemaphoreType.DMA,
    ],
)(_sc_gather_body)


def _tc_layer_body(agg_ref, xa_ref, wrel_ref, wroot_ref, b_ref, out_ref,
                   *, relu_pad):
    x = xa_ref[:, :D]
    acc = jnp.dot(x, wroot_ref[:], preferred_element_type=jnp.float32)
    acc = acc + b_ref[:]
    for r in range(R):
        a = agg_ref[:, r * DP:r * DP + D]
        cnt = agg_ref[:, r * DP + D:r * DP + D + 1]
        mean = a * (1.0 / jnp.maximum(cnt, 1.0))
        acc = acc + jnp.dot(mean, wrel_ref[r],
                            preferred_element_type=jnp.float32)
    if relu_pad:
        out_ref[:, :D] = jnp.maximum(acc, 0.0)
        pad = jax.lax.broadcasted_iota(jnp.int32, (acc.shape[0], DP - D), 1)
        out_ref[:, D:] = jnp.where(pad == 0, 1.0, 0.0)
    else:
        out_ref[:] = acc


def _tc_layer(agg2d, xa, wrel, wroot, b, relu_pad):
    bn = 1000
    grid = N // bn
    out_d = DP if relu_pad else D
    return pl.pallas_call(
        functools.partial(_tc_layer_body, relu_pad=relu_pad),
        grid=(grid,),
        in_specs=[
            pl.BlockSpec((bn, R * DP), lambda i: (i, 0)),
            pl.BlockSpec((bn, DP), lambda i: (i, 0)),
            pl.BlockSpec((R, D, D), lambda i: (0, 0, 0)),
            pl.BlockSpec((D, D), lambda i: (0, 0)),
            pl.BlockSpec((1, D), lambda i: (0, 0)),
        ],
        out_specs=pl.BlockSpec((bn, out_d), lambda i: (i, 0)),
        out_shape=jax.ShapeDtypeStruct((N, out_d), jnp.float32),
    )(agg2d, xa, wrel, wroot, b)


def _tc_score_body(zh_ref, zt_ref, relw_ref, rid_ref, out_ref):
    zh = zh_ref[:]
    zt = zt_ref[:]
    rid = rid_ref[:]
    acc = jnp.zeros((P, 1), jnp.float32)
    for r in range(R):
        m = jnp.dot(zh, relw_ref[r], preferred_element_type=jnp.float32)
        s = jnp.sum(m * zt, axis=1, keepdims=True)
        acc = jnp.where(rid == r, s, acc)
    out_ref[:] = acc


def _tc_score(zh, zt, relw, rid2d):
    return pl.pallas_call(
        _tc_score_body,
        out_shape=jax.ShapeDtypeStruct((P, 1), jnp.float32),
    )(zh, zt, relw, rid2d)


def kernel(x0, W_rel1, W_root1, b1, W_rel2, W_root2, b2, rel_W,
           edge_index, edge_type, rel_ids, head, tail):
    src = edge_index[0].astype(jnp.int32)
    dst = edge_index[1].astype(jnp.int32)
    et = edge_type.astype(jnp.int32)

    ones_pad = jnp.concatenate(
        [jnp.ones((N, 1), jnp.float32), jnp.zeros((N, DP - D - 1), jnp.float32)],
        axis=1)
    xa0 = jnp.concatenate([x0, ones_pad], axis=1)

    agg1 = _sc_aggregate(xa0, src, dst, et)
    xa1 = _tc_layer(agg1.reshape(N, R * DP), xa0, W_rel1, W_root1,
                    b1.reshape(1, D), relu_pad=True)

    agg2 = _sc_aggregate(xa1, src, dst, et)
    z = _tc_layer(agg2.reshape(N, R * DP), xa1, W_rel2, W_root2,
                  b2.reshape(1, D), relu_pad=False)

    zh = _sc_gather(z, head.astype(jnp.int32))
    zt = _sc_gather(z, tail.astype(jnp.int32))

    scores = _tc_score(zh, zt, rel_W, rel_ids.reshape(P, 1).astype(jnp.int32))
    return scores.reshape(P)


# speedup vs baseline: 4.2629x; 4.2629x over previous
import functools

import jax
import jax.numpy as jnp
from jax import lax
from jax.experimental import pallas as pl
from jax.experimental.pallas import tpu as pltpu
from jax.experimental.pallas import tpu_sc as plsc

N = 10000
E = 320000
D = 128
R = 8
P = 8192

C = 640
NCH = 16
SEG = C * R
ACC_ROWS = 5248
K = 64
CW = 128

_NS = 16
_NC = 2
E_T = E // _NS
SL = 4000
NSL = E_T // SL
CAP = ((SL + K) + K - 1) // K * K

_SC_PARAMS = pltpu.CompilerParams(needs_layout_passes=False)


def _make_sc_agg_body(with_cnt):
  def _sc_agg_body(x_hbm, src_hbm, dst_hbm, et_hbm, z128_hbm, ones_hbm,
                   out_hbm, cnt_hbm,
                   src_v, dst_v, et_v, csrc, cseg, rows, ones_v, acc, cacc,
                   sem):
    c = lax.axis_index("c")
    s = lax.axis_index("s")
    base = s * E_T

    if with_cnt:
        pltpu.sync_copy(ones_hbm, ones_v)
    lanes = jnp.arange(16, dtype=jnp.int32)

    def _chunk(ch, _):
        chunk = ch * _NC + c
        nlo = chunk * C

        zb = s * (ACC_ROWS // _NS)
        for j in range(2):
            pltpu.sync_copy(z128_hbm, acc.at[pl.ds(zb + j * 128, 128)])
            if with_cnt:
                pltpu.sync_copy(z128_hbm, cacc.at[pl.ds(zb + j * 128, 128)])
        pltpu.sync_copy(z128_hbm.at[pl.ds(0, 72)],
                        acc.at[pl.ds(zb + 256, 72)])
        if with_cnt:
            pltpu.sync_copy(z128_hbm.at[pl.ds(0, 72)],
                            cacc.at[pl.ds(zb + 256, 72)])
        plsc.subcore_barrier()

        for q in range(NSL):
            qb = base + q * SL
            pltpu.sync_copy(src_hbm.at[pl.ds(qb, SL)], src_v)
            pltpu.sync_copy(dst_hbm.at[pl.ds(qb, SL)], dst_v)
            pltpu.sync_copy(et_hbm.at[pl.ds(qb, SL)], et_v)

            def _scan(i, ptr):
                sl = pl.ds(i * 16, 16)
                dv = dst_v[sl]
                m = (dv >= nlo) & (dv < nlo + C)
                sv = (dv - nlo) * R + et_v[sl]
                cum = plsc.cumsum(jnp.where(m, 1, 0))
                pos = ptr + cum - 1
                plsc.store_scatter(csrc, [pos], src_v[sl], mask=m)
                plsc.store_scatter(cseg, [pos >> 6, pos & 63], sv, mask=m)
                return ptr + cum[15]
            ptr = lax.fori_loop(0, SL // 16, _scan, 0)

            for j in range(K // 16):
                p = ptr + j * 16 + lanes
                plsc.store_scatter(csrc, [p], lanes * 0)
                plsc.store_scatter(cseg, [p >> 6, p & 63], lanes * 0 + SEG)

            def _batch(b, _):
                pltpu.async_copy(x_hbm.at[csrc.at[pl.ds(b * K, K)]], rows,
                                 sem).wait()
                pltpu.sync_copy(rows, acc.at[cseg.at[b]], add=True)
                if with_cnt:
                    pltpu.sync_copy(ones_v, cacc.at[cseg.at[b]], add=True)
                return 0
            lax.fori_loop(0, (ptr + K - 1) // K, _batch, 0)
        plsc.subcore_barrier()

        wb = s * (SEG // _NS)
        lo = chunk * SEG
        for j in range(2):
            pltpu.sync_copy(acc.at[pl.ds(wb + j * 128, 128)],
                            out_hbm.at[pl.ds(lo + wb + j * 128, 128)])
            if with_cnt:
                pltpu.sync_copy(cacc.at[pl.ds(wb + j * 128, 128)],
                                cnt_hbm.at[pl.ds(lo + wb + j * 128, 128)])
        pltpu.sync_copy(acc.at[pl.ds(wb + 256, 64)],
                        out_hbm.at[pl.ds(lo + wb + 256, 64)])
        if with_cnt:
            pltpu.sync_copy(cacc.at[pl.ds(wb + 256, 64)],
                            cnt_hbm.at[pl.ds(lo + wb + 256, 64)])
        plsc.subcore_barrier()
        return 0

    lax.fori_loop(0, NCH // _NC, _chunk, 0)
  return _sc_agg_body


_sc_aggregate_cnt = functools.partial(
    pl.kernel,
    out_type=(jax.ShapeDtypeStruct((NCH * SEG, D), jnp.float32),
              jax.ShapeDtypeStruct((NCH * SEG, CW), jnp.float32)),
    mesh=plsc.VectorSubcoreMesh(core_axis_name="c", subcore_axis_name="s"),
    scratch_types=[
        pltpu.VMEM((SL,), jnp.int32),
        pltpu.VMEM((SL,), jnp.int32),
        pltpu.VMEM((SL,), jnp.int32),
        pltpu.VMEM((CAP,), jnp.int32),
        pltpu.VMEM((CAP // K, K), jnp.int32),
        pltpu.VMEM((K, D), jnp.float32),
        pltpu.VMEM((K, CW), jnp.float32),
        pltpu.VMEM_SHARED((ACC_ROWS, D), jnp.float32),
        pltpu.VMEM_SHARED((ACC_ROWS, CW), jnp.float32),
        pltpu.SemaphoreType.DMA,
    ],
    compiler_params=_SC_PARAMS,
)(_make_sc_agg_body(True))


def _sc_agg_nc_body(x_hbm, src_hbm, dst_hbm, et_hbm, z128_hbm, out_hbm,
                    src_v, dst_v, et_v, csrc, cseg, rows, acc, sem):
    body = _make_sc_agg_body(False)
    body(x_hbm, src_hbm, dst_hbm, et_hbm, z128_hbm, None, out_hbm, None,
         src_v, dst_v, et_v, csrc, cseg, rows, None, acc, None, sem)


_sc_aggregate_nc = functools.partial(
    pl.kernel,
    out_type=jax.ShapeDtypeStruct((NCH * SEG, D), jnp.float32),
    mesh=plsc.VectorSubcoreMesh(core_axis_name="c", subcore_axis_name="s"),
    scratch_types=[
        pltpu.VMEM((SL,), jnp.int32),
        pltpu.VMEM((SL,), jnp.int32),
        pltpu.VMEM((SL,), jnp.int32),
        pltpu.VMEM((CAP,), jnp.int32),
        pltpu.VMEM((CAP // K, K), jnp.int32),
        pltpu.VMEM((K, D), jnp.float32),
        pltpu.VMEM_SHARED((ACC_ROWS, D), jnp.float32),
        pltpu.SemaphoreType.DMA,
    ],
    compiler_params=_SC_PARAMS,
)(_sc_agg_nc_body)


def _sc_gather_body(tab_hbm, idx_hbm, out_hbm, idx_v, rows_v, sem):
    wid = lax.axis_index("s") * _NC + lax.axis_index("c")
    b_per_w = P // (_NC * _NS)
    base = wid * b_per_w
    pltpu.sync_copy(idx_hbm.at[pl.ds(base, b_per_w)], idx_v)
    pltpu.async_copy(tab_hbm.at[idx_v], rows_v, sem).wait()
    pltpu.sync_copy(rows_v, out_hbm.at[pl.ds(base, b_per_w)])


_sc_gather = functools.partial(
    pl.kernel,
    out_type=jax.ShapeDtypeStruct((P, D), jnp.float32),
    mesh=plsc.VectorSubcoreMesh(core_axis_name="c", subcore_axis_name="s"),
    scratch_types=[
        pltpu.VMEM((P // (_NC * _NS),), jnp.int32),
        pltpu.VMEM((P // (_NC * _NS), D), jnp.float32),
        pltpu.SemaphoreType.DMA,
    ],
    compiler_params=_SC_PARAMS,
)(_sc_gather_body)


def _tc_layer_body(agg_ref, cnt_ref, x_ref, wrel_ref, wroot_ref, b_ref,
                   out_ref, *, relu):
    acc = jnp.dot(x_ref[:], wroot_ref[:], preferred_element_type=jnp.float32)
    acc = acc + b_ref[:]
    for r in range(R):
        a = agg_ref[:, r * D:(r + 1) * D]
        cnt = cnt_ref[:, r * CW:r * CW + 1]
        mean = a * (1.0 / jnp.maximum(cnt, 1.0))
        acc = acc + jnp.dot(mean, wrel_ref[r],
                            preferred_element_type=jnp.float32)
    if relu:
        acc = jnp.maximum(acc, 0.0)
    out_ref[:] = acc


def _tc_layer(agg2d, cnt2d, x, wrel, wroot, b, relu):
    bn = 1000
    grid = N // bn
    return pl.pallas_call(
        functools.partial(_tc_layer_body, relu=relu),
        grid=(grid,),
        in_specs=[
            pl.BlockSpec((bn, R * D), lambda i: (i, 0)),
            pl.BlockSpec((bn, R * CW), lambda i: (i, 0)),
            pl.BlockSpec((bn, D), lambda i: (i, 0)),
            pl.BlockSpec((R, D, D), lambda i: (0, 0, 0)),
            pl.BlockSpec((D, D), lambda i: (0, 0)),
            pl.BlockSpec((1, D), lambda i: (0, 0)),
        ],
        out_specs=pl.BlockSpec((bn, D), lambda i: (i, 0)),
        out_shape=jax.ShapeDtypeStruct((N, D), jnp.float32),
    )(agg2d, cnt2d, x, wrel, wroot, b)


def _tc_score_body(zh_ref, zt_ref, relw_ref, rid_ref, out_ref):
    zh = zh_ref[:]
    zt = zt_ref[:]
    rid = rid_ref[:]
    acc = jnp.zeros((P, 1), jnp.float32)
    for r in range(R):
        m = jnp.dot(zh, relw_ref[r], preferred_element_type=jnp.float32)
        s = jnp.sum(m * zt, axis=1, keepdims=True)
        acc = jnp.where(rid == r, s, acc)
    out_ref[:] = acc


def _tc_score(zh, zt, relw, rid2d):
    return pl.pallas_call(
        _tc_score_body,
        out_shape=jax.ShapeDtypeStruct((P, 1), jnp.float32),
    )(zh, zt, relw, rid2d)


def kernel(x0, W_rel1, W_root1, b1, W_rel2, W_root2, b2, rel_W,
           edge_index, edge_type, rel_ids, head, tail):
    src = edge_index[0].astype(jnp.int32)
    dst = edge_index[1].astype(jnp.int32)
    et = edge_type.astype(jnp.int32)

    z128 = jnp.zeros((128, D), jnp.float32)
    ones = jnp.ones((K, CW), jnp.float32)

    agg1, cnt1 = _sc_aggregate_cnt(x0, src, dst, et, z128, ones)
    cnt2d = cnt1[:N * R].reshape(N, R * CW)
    x1 = _tc_layer(agg1[:N * R].reshape(N, R * D), cnt2d,
                   x0, W_rel1, W_root1, b1.reshape(1, D), relu=True)

    agg2 = _sc_aggregate_nc(x1, src, dst, et, z128)
    z = _tc_layer(agg2[:N * R].reshape(N, R * D), cnt2d,
                  x1, W_rel2, W_root2, b2.reshape(1, D), relu=False)

    zh = _sc_gather(z, head.astype(jnp.int32))
    zt = _sc_gather(z, tail.astype(jnp.int32))

    scores = _tc_score(zh, zt, rel_W, rel_ids.reshape(P, 1).astype(jnp.int32))
    return scores.reshape(P)
